# baseline (speedup 1.0000x reference)
import functools

import jax
import jax.numpy as jnp
from jax import lax
from jax.experimental import pallas as pl
from jax.experimental.pallas import tpu as pltpu
from jax.experimental.pallas import tpu_sc as plsc

_NC, _NS, _L = 2, 16, 16

_DNUMS = lax.GatherDimensionNumbers(
    offset_dims=(), collapsed_slice_dims=(0,), start_index_map=(0,))


def _dg(v, idx):
    return lax.gather(v, idx[:, None], _DNUMS, (1,),
                      mode=lax.GatherScatterMode.PROMISE_IN_BOUNDS)


def _ifloor(v):
    t = v.astype(jnp.int32)
    return jnp.where(t.astype(jnp.float32) > v, t - 1, t)


@functools.lru_cache(maxsize=None)
def _build(B, C, H, W, n_lm):
    nch = W // _L
    ch_rows = (2 * n_lm + 7) // 8 * 8
    rows_per_b = C * ch_rows
    mesh = plsc.VectorSubcoreMesh(
        core_axis_name="c", subcore_axis_name="s",
        num_cores=_NC, num_subcores=_NS)

    @functools.partial(
        pl.kernel,
        out_type=jax.ShapeDtypeStruct((B * rows_per_b, 128), jnp.float32),
        mesh=mesh,
        scratch_types=[
            pltpu.VMEM((H, W), jnp.float32),
            pltpu.VMEM((H, W), jnp.float32),
            pltpu.VMEM((n_lm * _L,), jnp.float32),
            pltpu.VMEM((n_lm * _L,), jnp.float32),
            pltpu.VMEM((ch_rows, 128), jnp.float32),
            pltpu.SemaphoreType.DMA,
            pltpu.SemaphoreType.DMA,
            pltpu.SemaphoreType.DMA,
        ],
    )
    def kern(img_hbm, xbc_hbm, ybc_hbm, out_hbm, plane0, plane1, xv, yv,
             slab, sem_a, sem_b, sem_o):
        b = lax.axis_index("s") * _NC + lax.axis_index("c")
        pltpu.sync_copy(xbc_hbm.at[pl.ds(b * n_lm * _L, n_lm * _L)], xv)
        pltpu.sync_copy(ybc_hbm.at[pl.ds(b * n_lm * _L, n_lm * _L)], yv)
        iota = lax.iota(jnp.int32, _L)
        planes = (plane0, plane1)
        sems = (sem_a, sem_b)

        h_plane = [None, None]
        h_plane[0] = pltpu.async_copy(img_hbm.at[b, 0], plane0, sems[0])
        h_out = None

        for ch in range(C):
            h_plane[ch % 2].wait()
            if ch + 1 < C:
                h_plane[(ch + 1) % 2] = pltpu.async_copy(
                    img_hbm.at[b, ch + 1], planes[(ch + 1) % 2],
                    sems[(ch + 1) % 2])
            plane = planes[ch % 2]
            if h_out is not None:
                h_out.wait()

            @plsc.parallel_loop(0, n_lm, unroll=4)
            def lm_body(i):
                off = pl.multiple_of(i * _L, 16)
                axv = xv[pl.ds(off, _L)]
                ayv = yv[pl.ds(off, _L)]
                bxv = _ifloor(axv)
                byv = _ifloor(ayv)
                fxv = axv - bxv.astype(jnp.float32)
                fyv = ayv - byv.astype(jnp.float32)
                bx_s = bxv[0]
                by_s = byv[0]

                ca = jnp.clip((bx_s - 15) >> 4, 0, nch - 1)
                s0 = pl.multiple_of(ca * 16, 16)
                s1 = pl.multiple_of(jnp.minimum(ca + 1, nch - 1) * 16, 16)
                cav = jnp.clip((bxv - 15) >> 4, 0, nch - 1) * 16

                col0 = bxv - iota
                col1 = col0 + 1
                lc0 = col0 - cav
                lc1 = col1 - cav
                a0 = jnp.where((col0 >= 0) & (col0 <= W - 1), 1.0 - fxv, 0.0)
                a1 = jnp.where((col1 >= 0) & (col1 <= W - 1), fxv, 0.0)
                sel0 = lc0 < _L
                sel1 = lc1 < _L
                i0 = lc0 & (_L - 1)
                i1 = lc1 & (_L - 1)

                def rowcomb(r):
                    rg = jnp.clip(by_s + r, 0, H - 1)
                    v0 = plane[rg, pl.ds(s0, _L)]
                    v1 = plane[rg, pl.ds(s1, _L)]
                    g0 = jnp.where(sel0, _dg(v0, i0), _dg(v1, i0))
                    g1 = jnp.where(sel1, _dg(v0, i1), _dg(v1, i1))
                    rw = jnp.where((byv + r >= 0) & (byv + r <= H - 1),
                                   1.0, 0.0)
                    return (a0 * g0 + a1 * g1) * rw

                rc_prev = rowcomb(0)
                for pi in range(16):
                    rc_cur = rowcomb(pi + 1)
                    slab[2 * i + pi // 8, pl.ds((pi % 8) * 16, _L)] = (
                        (1.0 - fyv) * rc_prev + fyv * rc_cur)
                    rc_prev = rc_cur

            h_out = pltpu.async_copy(
                slab,
                out_hbm.at[pl.ds(b * rows_per_b + ch * ch_rows, ch_rows)],
                sem_o)
        h_out.wait()

    return kern


def kernel(batch, landmarks, patch_size):
    B, C, H, W = batch.shape
    n_lm = landmarks.shape[1] // 2
    half = patch_size / 2.0
    lm = landmarks.reshape(B, n_lm, 2)
    ax = lm[..., 0].astype(jnp.float32) + (half - 0.5)
    ay = lm[..., 1].astype(jnp.float32) - (half + 0.5)
    xbc = jnp.broadcast_to(ax[..., None], (B, n_lm, _L)).reshape(-1)
    ybc = jnp.broadcast_to(ay[..., None], (B, n_lm, _L)).reshape(-1)
    out = _build(B, C, H, W, n_lm)(batch, xbc, ybc)
    ch_rows = (2 * n_lm + 7) // 8 * 8
    out = out.reshape(B, C, ch_rows // 2, 16, 16).transpose(0, 2, 1, 3, 4)
    return out[:, :n_lm]


# speedup vs baseline: 1.5164x; 1.5164x over previous
import functools

import jax
import jax.numpy as jnp
from jax import lax
from jax.experimental import pallas as pl
from jax.experimental.pallas import tpu as pltpu
from jax.experimental.pallas import tpu_sc as plsc

_NC, _NS, _L = 2, 16, 16

_DNUMS = lax.GatherDimensionNumbers(
    offset_dims=(), collapsed_slice_dims=(0,), start_index_map=(0,))


def _dg(v, idx):
    return lax.gather(v, idx[:, None], _DNUMS, (1,),
                      mode=lax.GatherScatterMode.PROMISE_IN_BOUNDS)


def _ifloor(v):
    t = v.astype(jnp.int32)
    return jnp.where(t.astype(jnp.float32) > v, t - 1, t)


@functools.lru_cache(maxsize=None)
def _build(B, C, H, W, n_lm):
    nch = W // _L
    ch_rows = (2 * n_lm + 7) // 8 * 8
    rows_per_b = C * ch_rows
    mesh = plsc.VectorSubcoreMesh(
        core_axis_name="c", subcore_axis_name="s",
        num_cores=_NC, num_subcores=_NS)

    @functools.partial(
        pl.kernel,
        out_type=jax.ShapeDtypeStruct((B * rows_per_b, 128), jnp.float32),
        mesh=mesh,
        scratch_types=[
            pltpu.VMEM((H, W), jnp.float32),
            pltpu.VMEM((H, W), jnp.float32),
            pltpu.VMEM((n_lm * _L,), jnp.float32),
            pltpu.VMEM((n_lm * _L,), jnp.float32),
            pltpu.VMEM((ch_rows, 128), jnp.float32),
            pltpu.SemaphoreType.DMA,
            pltpu.SemaphoreType.DMA,
            pltpu.SemaphoreType.DMA,
        ],
    )
    def kern(img_hbm, xbc_hbm, ybc_hbm, out_hbm, plane0, plane1, xv, yv,
             slab, sem_a, sem_b, sem_o):
        b = lax.axis_index("s") * _NC + lax.axis_index("c")
        pltpu.sync_copy(xbc_hbm.at[pl.ds(b * n_lm * _L, n_lm * _L)], xv)
        pltpu.sync_copy(ybc_hbm.at[pl.ds(b * n_lm * _L, n_lm * _L)], yv)
        iota = lax.iota(jnp.int32, _L)
        planes = (plane0, plane1)
        sems = (sem_a, sem_b)

        h_plane = [None, None]
        h_plane[0] = pltpu.async_copy(img_hbm.at[b, 0], plane0, sems[0])
        h_out = None

        for ch in range(C):
            h_plane[ch % 2].wait()
            if ch + 1 < C:
                h_plane[(ch + 1) % 2] = pltpu.async_copy(
                    img_hbm.at[b, ch + 1], planes[(ch + 1) % 2],
                    sems[(ch + 1) % 2])
            plane = planes[ch % 2]
            if h_out is not None:
                h_out.wait()

            @plsc.parallel_loop(0, n_lm, unroll=4)
            def lm_body(i):
                off = pl.multiple_of(i * _L, 16)
                axv = xv[pl.ds(off, _L)]
                ayv = yv[pl.ds(off, _L)]
                bxv = _ifloor(axv)
                byv = _ifloor(ayv)
                fxv = axv - bxv.astype(jnp.float32)
                fyv = ayv - byv.astype(jnp.float32)
                bx_s = bxv[0]
                by_s = byv[0]

                ca = jnp.clip((bx_s - 15) >> 4, 0, nch - 1)
                s0 = pl.multiple_of(ca * 16, 16)
                s1 = pl.multiple_of(jnp.minimum(ca + 1, nch - 1) * 16, 16)
                cav = jnp.clip((bxv - 15) >> 4, 0, nch - 1) * 16

                col0 = bxv - iota
                col1 = col0 + 1
                lc0 = col0 - cav
                lc1 = col1 - cav
                a0 = jnp.where((col0 >= 0) & (col0 <= W - 1), 1.0 - fxv, 0.0)
                a1 = jnp.where((col1 >= 0) & (col1 <= W - 1), fxv, 0.0)
                sel0 = lc0 < _L
                sel1 = lc1 < _L
                i0 = lc0 & (_L - 1)
                i1 = lc1 & (_L - 1)

                def rowcomb(r):
                    rg = jnp.clip(by_s + r, 0, H - 1)
                    v0 = plane[rg, pl.ds(s0, _L)]
                    v1 = plane[rg, pl.ds(s1, _L)]
                    g0 = jnp.where(sel0, _dg(v0, i0), _dg(v1, i0))
                    g1 = jnp.where(sel1, _dg(v0, i1), _dg(v1, i1))
                    rw = jnp.where((byv + r >= 0) & (byv + r <= H - 1),
                                   1.0, 0.0)
                    return (a0 * g0 + a1 * g1) * rw

                rc_prev = rowcomb(0)
                for pi in range(16):
                    rc_cur = rowcomb(pi + 1)
                    slab[2 * i + pi // 8, pl.ds((pi % 8) * 16, _L)] = (
                        (1.0 - fyv) * rc_prev + fyv * rc_cur)
                    rc_prev = rc_cur

            h_out = pltpu.async_copy(
                slab,
                out_hbm.at[pl.ds(b * rows_per_b + ch * ch_rows, ch_rows)],
                sem_o)
        h_out.wait()

    return kern


def kernel(batch, landmarks, patch_size):
    B, C, H, W = batch.shape
    n_lm = landmarks.shape[1] // 2
    half = patch_size / 2.0
    lm = landmarks.reshape(B, n_lm, 2)
    ax = lm[..., 0].astype(jnp.float32) + (half - 0.5)
    ay = lm[..., 1].astype(jnp.float32) - (half + 0.5)
    xbc = jnp.broadcast_to(ax[..., None], (B, n_lm, _L)).reshape(-1)
    ybc = jnp.broadcast_to(ay[..., None], (B, n_lm, _L)).reshape(-1)
    out = _build(B, C, H, W, n_lm)(batch, xbc, ybc)
    ch_rows = (2 * n_lm + 7) // 8 * 8
    out = out.reshape(B, C, ch_rows, 128)[:, :, : 2 * n_lm]
    return out.reshape(B, C, n_lm, 16, 16).transpose(0, 2, 1, 3, 4)


# speedup vs baseline: 1.5424x; 1.0171x over previous
import functools

import jax
import jax.numpy as jnp
from jax import lax
from jax.experimental import pallas as pl
from jax.experimental.pallas import tpu as pltpu
from jax.experimental.pallas import tpu_sc as plsc

_NC, _NS, _L = 2, 16, 16

_DNUMS = lax.GatherDimensionNumbers(
    offset_dims=(), collapsed_slice_dims=(0,), start_index_map=(0,))


def _dg(v, idx):
    return lax.gather(v, idx[:, None], _DNUMS, (1,),
                      mode=lax.GatherScatterMode.PROMISE_IN_BOUNDS)


def _ifloor(v):
    t = v.astype(jnp.int32)
    return jnp.where(t.astype(jnp.float32) > v, t - 1, t)


@functools.lru_cache(maxsize=None)
def _build(B, C, H, W, n_lm):
    nch = W // _L
    ch_rows = (2 * n_lm + 7) // 8 * 8
    rows_per_b = C * ch_rows
    mesh = plsc.VectorSubcoreMesh(
        core_axis_name="c", subcore_axis_name="s",
        num_cores=_NC, num_subcores=_NS)

    @functools.partial(
        pl.kernel,
        out_type=jax.ShapeDtypeStruct((B * rows_per_b, 128), jnp.float32),
        mesh=mesh,
        scratch_types=[
            pltpu.VMEM((H, W), jnp.float32),
            pltpu.VMEM((H, W), jnp.float32),
            pltpu.VMEM((n_lm * _L,), jnp.float32),
            pltpu.VMEM((n_lm * _L,), jnp.float32),
            pltpu.VMEM((ch_rows, 128), jnp.float32),
            pltpu.SemaphoreType.DMA,
            pltpu.SemaphoreType.DMA,
            pltpu.SemaphoreType.DMA,
        ],
    )
    def kern(img_hbm, xbc_hbm, ybc_hbm, out_hbm, plane0, plane1, xv, yv,
             slab, sem_a, sem_b, sem_o):
        b = lax.axis_index("s") * _NC + lax.axis_index("c")
        pltpu.sync_copy(xbc_hbm.at[pl.ds(b * n_lm * _L, n_lm * _L)], xv)
        pltpu.sync_copy(ybc_hbm.at[pl.ds(b * n_lm * _L, n_lm * _L)], yv)
        iota = lax.iota(jnp.int32, _L)
        planes = (plane0, plane1)
        sems = (sem_a, sem_b)

        h_plane = [None, None]
        h_plane[0] = pltpu.async_copy(img_hbm.at[b, 0], plane0, sems[0])
        h_out = None

        for ch in range(C):
            h_plane[ch % 2].wait()
            if ch + 1 < C:
                h_plane[(ch + 1) % 2] = pltpu.async_copy(
                    img_hbm.at[b, ch + 1], planes[(ch + 1) % 2],
                    sems[(ch + 1) % 2])
            plane = planes[ch % 2]
            if h_out is not None:
                h_out.wait()

            @plsc.parallel_loop(0, n_lm, unroll=7)
            def lm_body(i):
                off = pl.multiple_of(i * _L, 16)
                axv = xv[pl.ds(off, _L)]
                ayv = yv[pl.ds(off, _L)]
                bxv = _ifloor(axv)
                byv = _ifloor(ayv)
                fxv = axv - bxv.astype(jnp.float32)
                fyv = ayv - byv.astype(jnp.float32)
                bx_s = bxv[0]
                by_s = byv[0]

                ca = jnp.clip((bx_s - 15) >> 4, 0, nch - 1)
                s0 = pl.multiple_of(ca * 16, 16)
                s1 = pl.multiple_of(jnp.minimum(ca + 1, nch - 1) * 16, 16)
                cav = jnp.clip((bxv - 15) >> 4, 0, nch - 1) * 16

                col0 = bxv - iota
                col1 = col0 + 1
                lc0 = col0 - cav
                lc1 = col1 - cav
                a0 = jnp.where((col0 >= 0) & (col0 <= W - 1), 1.0 - fxv, 0.0)
                a1 = jnp.where((col1 >= 0) & (col1 <= W - 1), fxv, 0.0)
                sel0 = lc0 < _L
                sel1 = lc1 < _L
                i0 = lc0 & (_L - 1)
                i1 = lc1 & (_L - 1)

                def rowcomb(r):
                    rg = jnp.clip(by_s + r, 0, H - 1)
                    v0 = plane[rg, pl.ds(s0, _L)]
                    v1 = plane[rg, pl.ds(s1, _L)]
                    g0 = jnp.where(sel0, _dg(v0, i0), _dg(v1, i0))
                    g1 = jnp.where(sel1, _dg(v0, i1), _dg(v1, i1))
                    rw = jnp.where((byv + r >= 0) & (byv + r <= H - 1),
                                   1.0, 0.0)
                    return (a0 * g0 + a1 * g1) * rw

                rc_prev = rowcomb(0)
                for pi in range(16):
                    rc_cur = rowcomb(pi + 1)
                    slab[2 * i + pi // 8, pl.ds((pi % 8) * 16, _L)] = (
                        (1.0 - fyv) * rc_prev + fyv * rc_cur)
                    rc_prev = rc_cur

            h_out = pltpu.async_copy(
                slab,
                out_hbm.at[pl.ds(b * rows_per_b + ch * ch_rows, ch_rows)],
                sem_o)
        h_out.wait()

    return kern


def kernel(batch, landmarks, patch_size):
    B, C, H, W = batch.shape
    n_lm = landmarks.shape[1] // 2
    half = patch_size / 2.0
    lm = landmarks.reshape(B, n_lm, 2)
    ax = lm[..., 0].astype(jnp.float32) + (half - 0.5)
    ay = lm[..., 1].astype(jnp.float32) - (half + 0.5)
    xbc = jnp.broadcast_to(ax[..., None], (B, n_lm, _L)).reshape(-1)
    ybc = jnp.broadcast_to(ay[..., None], (B, n_lm, _L)).reshape(-1)
    out = _build(B, C, H, W, n_lm)(batch, xbc, ybc)
    ch_rows = (2 * n_lm + 7) // 8 * 8
    out = out.reshape(B, C, ch_rows, 128)[:, :, : 2 * n_lm]
    return out.reshape(B, C, n_lm, 16, 16).transpose(0, 2, 1, 3, 4)
